# baseline (speedup 1.0000x reference)
import functools

import jax
import jax.numpy as jnp
from jax import lax
from jax.experimental import pallas as pl
from jax.experimental.pallas import tpu as pltpu
from jax.experimental.pallas import tpu_sc as plsc

N_NODES = 10000
N_EDGES = 320000
D_IN = 128
D_HID = 256
D_OUT = 128
N_GRAPHS = 64

NT = 32
DPT = 320
NP = NT * DPT
EBLK = 2000
NBLK_E = N_EDGES // EBLK
CHUNK = 64
CAP = N_EDGES + NBLK_E * 16 + 128
IBLK = 1024
DPW = 128
BM = 1024
GRID = NP // BM


def _prep_body(x_ref, wi1, bi1, wi2, bi2, wp1, bp1, wp2, bp2, wl2, wl1,
               q_ref, pp_ref):
    xb = x_ref[...]
    h0 = jnp.maximum(xb @ wi1[...] + bi1[...], 0.0) @ wi2[...] + bi2[...]
    p1 = jnp.maximum(h0 @ wp1[...] + bp1[...], 0.0)
    pos = p1 @ wp2[...] + bp2[...]
    pp = pos @ wl2[...]
    pp_ref[...] = pp
    q_ref[...] = h0 @ wl1[...] + pp


def _layer_body(m_ref, pp_ref, wg, bg, bl, wl1, h_ref, q_ref):
    t = m_ref[...] - pp_ref[...] + bl[...]
    h = jnp.maximum(t @ wg[...] + bg[...], 0.0)
    h_ref[...] = h
    q_ref[...] = h @ wl1[...] + pp_ref[...]


def _pool_body(h_ref, b_ref, wf, bf, out_ref, acc):
    i = pl.program_id(0)

    @pl.when(i == 0)
    def _():
        acc[...] = jnp.zeros_like(acc)

    b = b_ref[0]
    g = lax.broadcasted_iota(jnp.int32, (N_GRAPHS, BM), 0)
    oh = (b == g).astype(jnp.float32)
    acc[...] += oh @ h_ref[...]

    @pl.when(i == GRID - 1)
    def _():
        out_ref[...] = acc[...] @ wf[...] + bf[...]


def _full(a):
    return pl.BlockSpec(a.shape, lambda i: (0,) * a.ndim)


def _tc_prep(xp, wi1, bi1, wi2, bi2, wp1, bp1, wp2, bp2, wl2, wl1):
    ws = [wi1, bi1, wi2, bi2, wp1, bp1, wp2, bp2, wl2, wl1]
    return pl.pallas_call(
        _prep_body,
        grid=(GRID,),
        in_specs=[pl.BlockSpec((BM, D_IN), lambda i: (i, 0))] + [_full(w) for w in ws],
        out_specs=[pl.BlockSpec((BM, D_HID), lambda i: (i, 0))] * 2,
        out_shape=[jax.ShapeDtypeStruct((NP, D_HID), jnp.float32)] * 2,
    )(xp, *ws)


def _tc_layer(m, pp, wg, bg, bl, wl1):
    ws = [wg, bg, bl, wl1]
    return pl.pallas_call(
        _layer_body,
        grid=(GRID,),
        in_specs=[pl.BlockSpec((BM, D_HID), lambda i: (i, 0))] * 2 + [_full(w) for w in ws],
        out_specs=[pl.BlockSpec((BM, D_HID), lambda i: (i, 0))] * 2,
        out_shape=[jax.ShapeDtypeStruct((NP, D_HID), jnp.float32)] * 2,
    )(m, pp, *ws)


def _tc_pool(h, b3, wf, bf):
    return pl.pallas_call(
        _pool_body,
        grid=(GRID,),
        in_specs=[pl.BlockSpec((BM, D_HID), lambda i: (i, 0)),
                  pl.BlockSpec((1, 1, BM), lambda i: (i, 0, 0)),
                  _full(wf), _full(bf)],
        out_specs=pl.BlockSpec((N_GRAPHS, D_OUT), lambda i: (0, 0)),
        out_shape=jax.ShapeDtypeStruct((N_GRAPHS, D_OUT), jnp.float32),
        scratch_shapes=[pltpu.VMEM((N_GRAPHS, D_HID), jnp.float32)],
        compiler_params=pltpu.CompilerParams(
            dimension_semantics=("arbitrary",)),
    )(h, b3, wf, bf)


def _bucket_body(src_hbm, dst_hbm, plist, counts, sv0, dv0, sv1, dv1, ss, cs,
                 sem0, sem1):
    wid = lax.axis_index("c") * 16 + lax.axis_index("s")
    lo = wid * DPT
    hi = lo + DPT

    def issue(b, sv, dv, sem):
        pltpu.async_copy(src_hbm.at[pl.ds(b * EBLK, EBLK)], sv, sem)
        pltpu.async_copy(dst_hbm.at[pl.ds(b * EBLK, EBLK)], dv, sem)

    def wait(b, sv, dv, sem):
        pltpu.make_async_copy(src_hbm.at[pl.ds(b * EBLK, EBLK)], sv, sem).wait()
        pltpu.make_async_copy(dst_hbm.at[pl.ds(b * EBLK, EBLK)], dv, sem).wait()

    def process(sv, dv, gc):
        def ch(c, off):
            s16 = sv[pl.ds(c * 16, 16)]
            d16 = dv[pl.ds(c * 16, 16)]
            m = (d16 >= lo) & (d16 < hi)
            packed = (s16 << 9) | (d16 - lo)
            plsc.store_compressed(ss.at[pl.ds(off, 16)], packed, mask=m)
            pc = plsc.all_reduce_population_count(m)
            return off + pc[0]

        off = lax.fori_loop(0, EBLK // 16, ch, 0)
        ss[pl.ds(off, 16)] = jnp.full((16,), lo << 9, jnp.int32)
        offpad = ((off + 15) // 16) * 16
        base = pl.multiple_of(wid * CAP + gc, 16)
        pltpu.sync_copy(ss.at[pl.ds(0, EBLK + 16)],
                        plist.at[pl.ds(base, EBLK + 16)])
        return gc + offpad

    issue(0, sv0, dv0, sem0)

    def blkpair(p, gc):
        b0 = 2 * p
        issue(b0 + 1, sv1, dv1, sem1)
        wait(b0, sv0, dv0, sem0)
        gc = process(sv0, dv0, gc)

        @pl.when(b0 + 2 < NBLK_E)
        def _():
            issue(b0 + 2, sv0, dv0, sem0)

        wait(b0 + 1, sv1, dv1, sem1)
        gc = process(sv1, dv1, gc)
        return gc

    gc = lax.fori_loop(0, NBLK_E // 2, blkpair, 0)
    for k in range(128 // 16):
        ss[pl.ds(k * 16, 16)] = jnp.full((16,), lo << 9, jnp.int32)
    base = pl.multiple_of(wid * CAP + gc, 16)
    pltpu.sync_copy(ss.at[pl.ds(0, 128)], plist.at[pl.ds(base, 128)])
    padded = (gc // 128) * 128 + 128
    cs[...] = jnp.full((16,), padded, jnp.int32)
    pltpu.sync_copy(cs.at[pl.ds(0, 16)],
                    counts.at[pl.ds(pl.multiple_of(wid * 16, 16), 16)])


def _scmax_body(q_hbm, plist, counts, m_hbm, acc0, acc1, rows0, rows1, pbuf,
                sidx0, sidx1, dlb0, dlb1, cntv, sem0, sem1, sem2):
    wid = lax.axis_index("c") * 16 + lax.axis_index("s")
    lo = pl.multiple_of(wid * DPT, DPT)
    base = pl.multiple_of(wid * CAP, CHUNK)
    pltpu.sync_copy(q_hbm.at[pl.ds(lo, DPT)], acc0)
    pltpu.async_copy(q_hbm.at[pl.ds(lo, DPT)], acc1, sem2)
    pltpu.sync_copy(counts.at[pl.ds(pl.multiple_of(wid * 16, 16), 16)], cntv)
    n = cntv[pl.ds(0, 16)][0] // CHUNK

    def load_iblk(b):
        pltpu.sync_copy(plist.at[pl.ds(base + b * IBLK, IBLK)],
                        pbuf.at[pl.ds(0, IBLK)])

    def decode(c, sidx, dlb):
        o = (c % (IBLK // CHUNK)) * CHUNK
        for k in range(CHUNK // 16):
            v = pbuf[pl.ds(o + k * 16, 16)]
            sidx[pl.ds(k * 16, 16)] = v >> 9
            dlb[pl.ds(k * 16, 16)] = v & 511

    def rmw(accref, dl, rows, e):
        for j in range(DPW // 16):
            sl = pl.ds(j * 16, 16)
            a = plsc.bitcast(accref[dl, sl], jnp.bfloat16)
            r = plsc.bitcast(rows[e, sl], jnp.bfloat16)
            accref[dl, sl] = plsc.bitcast(jnp.maximum(a, r), jnp.float32)

    def process(rows, dlb):
        def quad(i, _):
            w = dlb[pl.ds(i * 4, 16)]
            rmw(acc0, w[0], rows, i * 4)
            rmw(acc1, w[1], rows, i * 4 + 1)
            rmw(acc0, w[2], rows, i * 4 + 2)
            rmw(acc1, w[3], rows, i * 4 + 3)
            return 0
        lax.fori_loop(0, CHUNK // 4, quad, 0)

    load_iblk(0)
    decode(0, sidx0, dlb0)
    pltpu.make_async_copy(q_hbm.at[pl.ds(lo, DPT)], acc1, sem2).wait()
    pltpu.async_copy(q_hbm.at[sidx0], rows0, sem0)

    def pair(p, _):
        c1 = 2 * p + 1
        decode(c1, sidx1, dlb1)
        pltpu.async_copy(q_hbm.at[sidx1], rows1, sem1)
        pltpu.make_async_copy(q_hbm.at[sidx0], rows0, sem0).wait()
        process(rows0, dlb0)
        c2 = 2 * p + 2

        @pl.when(c2 < n)
        def _():
            @pl.when(c2 % (IBLK // CHUNK) == 0)
            def _():
                load_iblk(c2 // (IBLK // CHUNK))
            decode(c2, sidx0, dlb0)
            pltpu.async_copy(q_hbm.at[sidx0], rows0, sem0)

        pltpu.make_async_copy(q_hbm.at[sidx1], rows1, sem1).wait()
        process(rows1, dlb1)
        return 0

    lax.fori_loop(0, n // 2, pair, 0)

    def mrow(r, _):
        for j in range(DPW // 16):
            sl = pl.ds(j * 16, 16)
            a = plsc.bitcast(acc0[r, sl], jnp.bfloat16)
            b = plsc.bitcast(acc1[r, sl], jnp.bfloat16)
            acc0[r, sl] = plsc.bitcast(jnp.maximum(a, b), jnp.float32)
        return 0

    lax.fori_loop(0, DPT, mrow, 0)
    pltpu.sync_copy(acc0, m_hbm.at[pl.ds(lo, DPT)])


@functools.cache
def _sc_kernels():
    mesh = plsc.VectorSubcoreMesh(core_axis_name="c", subcore_axis_name="s",
                                  num_cores=2, num_subcores=16)
    params = pltpu.CompilerParams(needs_layout_passes=False)
    bucket = pl.kernel(
        _bucket_body,
        out_type=(jax.ShapeDtypeStruct((NT * CAP,), jnp.int32),
                  jax.ShapeDtypeStruct((NT * 16,), jnp.int32)),
        mesh=mesh,
        compiler_params=params,
        scratch_types=[pltpu.VMEM((EBLK,), jnp.int32),
                       pltpu.VMEM((EBLK,), jnp.int32),
                       pltpu.VMEM((EBLK,), jnp.int32),
                       pltpu.VMEM((EBLK,), jnp.int32),
                       pltpu.VMEM((EBLK + 32,), jnp.int32),
                       pltpu.VMEM((16,), jnp.int32),
                       pltpu.SemaphoreType.DMA,
                       pltpu.SemaphoreType.DMA])
    scmax = pl.kernel(
        _scmax_body,
        out_type=jax.ShapeDtypeStruct((NP, DPW), jnp.float32),
        mesh=mesh,
        compiler_params=params,
        scratch_types=[pltpu.VMEM((DPT, DPW), jnp.float32),
                       pltpu.VMEM((DPT, DPW), jnp.float32),
                       pltpu.VMEM((CHUNK, DPW), jnp.float32),
                       pltpu.VMEM((CHUNK, DPW), jnp.float32),
                       pltpu.VMEM((IBLK + 16,), jnp.int32),
                       pltpu.VMEM((CHUNK,), jnp.int32),
                       pltpu.VMEM((CHUNK,), jnp.int32),
                       pltpu.VMEM((CHUNK + 16,), jnp.int32),
                       pltpu.VMEM((CHUNK + 16,), jnp.int32),
                       pltpu.VMEM((16,), jnp.int32),
                       pltpu.SemaphoreType.DMA,
                       pltpu.SemaphoreType.DMA,
                       pltpu.SemaphoreType.DMA])
    return bucket, scmax


def kernel(x, edge_index, batch, W_i1, b_i1, W_i2, b_i2, W_p1, b_p1,
           W_p2, b_p2, W_l, b_l, W_g, b_g, W_f, b_f):
    f32 = jnp.float32
    src = edge_index[0].astype(jnp.int32)
    dst = edge_index[1].astype(jnp.int32)
    xp = jnp.pad(x.astype(f32), ((0, NP - N_NODES), (0, 0)))
    bp = jnp.pad(batch.astype(jnp.int32), (0, NP - N_NODES),
                 constant_values=N_GRAPHS)

    wl1 = W_l[:D_HID]
    wl2 = jnp.pad(W_l[D_HID:], ((0, 128 - 3), (0, 0)))
    wp2 = jnp.pad(W_p2, ((0, 0), (0, 128 - 3)))
    bp2 = jnp.pad(b_p2, (0, 128 - 3)).reshape(1, 128)

    r = lambda v: v.reshape(1, -1)
    q, pp = _tc_prep(xp, W_i1, r(b_i1), W_i2, r(b_i2), W_p1, r(b_p1),
                     wp2, bp2, wl2, wl1)
    bucket, scmax = _sc_kernels()
    plist, counts = bucket(src, dst)

    def pack(v):
        vb = v.astype(jnp.bfloat16).reshape(NP, DPW, 2)
        return jax.lax.bitcast_convert_type(vb, jnp.float32)

    def unpack(vp):
        vb = jax.lax.bitcast_convert_type(vp, jnp.bfloat16)
        return vb.reshape(NP, D_HID).astype(jnp.float32)

    h = None
    for _ in range(3):
        mp = scmax(pack(q), plist, counts)
        h, q = _tc_layer(unpack(mp), pp, W_g, r(b_g), r(b_l), wl1)
    out = _tc_pool(h, bp.reshape(GRID, 1, BM), W_f, r(b_f))
    return out


# speedup vs baseline: 1.2172x; 1.2172x over previous
import functools

import jax
import jax.numpy as jnp
from jax import lax
from jax.experimental import pallas as pl
from jax.experimental.pallas import tpu as pltpu
from jax.experimental.pallas import tpu_sc as plsc

N_NODES = 10000
N_EDGES = 320000
D_IN = 128
D_HID = 256
D_OUT = 128
N_GRAPHS = 64

NT = 32
DPT = 320
NP = NT * DPT
EBLK = 2000
NBLK_E = N_EDGES // EBLK
CHUNK = 64
CAP = N_EDGES + NBLK_E * 16 + 128
IBLK = 1024
DPW = 128
BM = 1024
GRID = NP // BM


def _prep_body(x_ref, wi1, bi1, wi2, bi2, wp1, bp1, wp2, bp2, wl2, wl1,
               q_ref, pp_ref):
    xb = x_ref[...]
    h0 = jnp.maximum(xb @ wi1[...] + bi1[...], 0.0) @ wi2[...] + bi2[...]
    p1 = jnp.maximum(h0 @ wp1[...] + bp1[...], 0.0)
    pos = p1 @ wp2[...] + bp2[...]
    pp = pos @ wl2[...]
    pp_ref[...] = pp
    q_ref[...] = h0 @ wl1[...] + pp


def _layer_body(m_ref, pp_ref, wg, bg, bl, wl1, h_ref, q_ref):
    t = m_ref[...] - pp_ref[...] + bl[...]
    h = jnp.maximum(t @ wg[...] + bg[...], 0.0)
    h_ref[...] = h
    q_ref[...] = h @ wl1[...] + pp_ref[...]


def _pool_body(h_ref, b_ref, wf, bf, out_ref, acc):
    i = pl.program_id(0)

    @pl.when(i == 0)
    def _():
        acc[...] = jnp.zeros_like(acc)

    b = b_ref[0]
    g = lax.broadcasted_iota(jnp.int32, (N_GRAPHS, BM), 0)
    oh = (b == g).astype(jnp.float32)
    acc[...] += oh @ h_ref[...]

    @pl.when(i == GRID - 1)
    def _():
        out_ref[...] = acc[...] @ wf[...] + bf[...]


def _full(a):
    return pl.BlockSpec(a.shape, lambda i: (0,) * a.ndim)


def _tc_prep(xp, wi1, bi1, wi2, bi2, wp1, bp1, wp2, bp2, wl2, wl1):
    ws = [wi1, bi1, wi2, bi2, wp1, bp1, wp2, bp2, wl2, wl1]
    return pl.pallas_call(
        _prep_body,
        grid=(GRID,),
        in_specs=[pl.BlockSpec((BM, D_IN), lambda i: (i, 0))] + [_full(w) for w in ws],
        out_specs=[pl.BlockSpec((BM, D_HID), lambda i: (i, 0))] * 2,
        out_shape=[jax.ShapeDtypeStruct((NP, D_HID), jnp.float32)] * 2,
    )(xp, *ws)


def _tc_layer(m, pp, wg, bg, bl, wl1):
    ws = [wg, bg, bl, wl1]
    return pl.pallas_call(
        _layer_body,
        grid=(GRID,),
        in_specs=[pl.BlockSpec((BM, D_HID), lambda i: (i, 0))] * 2 + [_full(w) for w in ws],
        out_specs=[pl.BlockSpec((BM, D_HID), lambda i: (i, 0))] * 2,
        out_shape=[jax.ShapeDtypeStruct((NP, D_HID), jnp.float32)] * 2,
    )(m, pp, *ws)


def _tc_pool(h, b3, wf, bf):
    return pl.pallas_call(
        _pool_body,
        grid=(GRID,),
        in_specs=[pl.BlockSpec((BM, D_HID), lambda i: (i, 0)),
                  pl.BlockSpec((1, 1, BM), lambda i: (i, 0, 0)),
                  _full(wf), _full(bf)],
        out_specs=pl.BlockSpec((N_GRAPHS, D_OUT), lambda i: (0, 0)),
        out_shape=jax.ShapeDtypeStruct((N_GRAPHS, D_OUT), jnp.float32),
        scratch_shapes=[pltpu.VMEM((N_GRAPHS, D_HID), jnp.float32)],
        compiler_params=pltpu.CompilerParams(
            dimension_semantics=("arbitrary",)),
    )(h, b3, wf, bf)


def _bucket_body(src_hbm, dst_hbm, plist, counts, sv0, dv0, sv1, dv1, ss, cs,
                 sem0, sem1):
    wid = lax.axis_index("c") * 16 + lax.axis_index("s")
    lo = wid * DPT
    hi = lo + DPT

    def issue(b, sv, dv, sem):
        pltpu.async_copy(src_hbm.at[pl.ds(b * EBLK, EBLK)], sv, sem)
        pltpu.async_copy(dst_hbm.at[pl.ds(b * EBLK, EBLK)], dv, sem)

    def wait(b, sv, dv, sem):
        pltpu.make_async_copy(src_hbm.at[pl.ds(b * EBLK, EBLK)], sv, sem).wait()
        pltpu.make_async_copy(dst_hbm.at[pl.ds(b * EBLK, EBLK)], dv, sem).wait()

    def process(sv, dv, gc):
        def ch(c, off):
            s16 = sv[pl.ds(c * 16, 16)]
            d16 = dv[pl.ds(c * 16, 16)]
            m = (d16 >= lo) & (d16 < hi)
            packed = (s16 << 9) | (d16 - lo)
            plsc.store_compressed(ss.at[pl.ds(off, 16)], packed, mask=m)
            pc = plsc.all_reduce_population_count(m)
            return off + pc[0]

        off = lax.fori_loop(0, EBLK // 16, ch, 0)
        ss[pl.ds(off, 16)] = jnp.full((16,), lo << 9, jnp.int32)
        offpad = ((off + 15) // 16) * 16
        base = pl.multiple_of(wid * CAP + gc, 16)
        pltpu.sync_copy(ss.at[pl.ds(0, EBLK + 16)],
                        plist.at[pl.ds(base, EBLK + 16)])
        return gc + offpad

    issue(0, sv0, dv0, sem0)

    def blkpair(p, gc):
        b0 = 2 * p
        issue(b0 + 1, sv1, dv1, sem1)
        wait(b0, sv0, dv0, sem0)
        gc = process(sv0, dv0, gc)

        @pl.when(b0 + 2 < NBLK_E)
        def _():
            issue(b0 + 2, sv0, dv0, sem0)

        wait(b0 + 1, sv1, dv1, sem1)
        gc = process(sv1, dv1, gc)
        return gc

    gc = lax.fori_loop(0, NBLK_E // 2, blkpair, 0)
    for k in range(128 // 16):
        ss[pl.ds(k * 16, 16)] = jnp.full((16,), lo << 9, jnp.int32)
    base = pl.multiple_of(wid * CAP + gc, 16)
    pltpu.sync_copy(ss.at[pl.ds(0, 128)], plist.at[pl.ds(base, 128)])
    padded = (gc // 128) * 128 + 128
    cs[...] = jnp.full((16,), padded, jnp.int32)
    pltpu.sync_copy(cs.at[pl.ds(0, 16)],
                    counts.at[pl.ds(pl.multiple_of(wid * 16, 16), 16)])


def _scmax_body(q_hbm, plist, counts, m_hbm, acc0, acc1, rows0, rows1, pbuf,
                sidx0, sidx1, dlb0, dlb1, cntv, sem0, sem1, sem2):
    wid = lax.axis_index("c") * 16 + lax.axis_index("s")
    lo = pl.multiple_of(wid * DPT, DPT)
    base = pl.multiple_of(wid * CAP, CHUNK)
    pltpu.sync_copy(q_hbm.at[pl.ds(lo, DPT)], acc0)
    pltpu.async_copy(q_hbm.at[pl.ds(lo, DPT)], acc1, sem2)
    pltpu.sync_copy(counts.at[pl.ds(pl.multiple_of(wid * 16, 16), 16)], cntv)
    n = cntv[pl.ds(0, 16)][0] // CHUNK

    def load_iblk(b):
        pltpu.sync_copy(plist.at[pl.ds(base + b * IBLK, IBLK)],
                        pbuf.at[pl.ds(0, IBLK)])

    def decode(c, sidx, dlb):
        o = (c % (IBLK // CHUNK)) * CHUNK
        for k in range(CHUNK // 16):
            v = pbuf[pl.ds(o + k * 16, 16)]
            sidx[pl.ds(k * 16, 16)] = v >> 9
            dlb[pl.ds(k * 16, 16)] = v & 511

    def rmw2(dl0, dl1, rows, e0):
        for j in range(DPW // 16):
            sl = pl.ds(j * 16, 16)
            a0 = plsc.bitcast(acc0[dl0, sl], jnp.bfloat16)
            a1 = plsc.bitcast(acc1[dl1, sl], jnp.bfloat16)
            r0 = plsc.bitcast(rows[e0, sl], jnp.bfloat16)
            r1 = plsc.bitcast(rows[e0 + 1, sl], jnp.bfloat16)
            acc0[dl0, sl] = plsc.bitcast(jnp.maximum(a0, r0), jnp.float32)
            acc1[dl1, sl] = plsc.bitcast(jnp.maximum(a1, r1), jnp.float32)

    def process(rows, dlb):
        def quad(i, _):
            w = dlb[pl.ds(i * 4, 16)]
            rmw2(w[0], w[1], rows, i * 4)
            rmw2(w[2], w[3], rows, i * 4 + 2)
            return 0
        lax.fori_loop(0, CHUNK // 4, quad, 0)

    load_iblk(0)
    decode(0, sidx0, dlb0)
    pltpu.make_async_copy(q_hbm.at[pl.ds(lo, DPT)], acc1, sem2).wait()
    pltpu.async_copy(q_hbm.at[sidx0], rows0, sem0)

    def pair(p, _):
        c1 = 2 * p + 1
        decode(c1, sidx1, dlb1)
        pltpu.async_copy(q_hbm.at[sidx1], rows1, sem1)
        pltpu.make_async_copy(q_hbm.at[sidx0], rows0, sem0).wait()
        process(rows0, dlb0)
        c2 = 2 * p + 2

        @pl.when(c2 < n)
        def _():
            @pl.when(c2 % (IBLK // CHUNK) == 0)
            def _():
                load_iblk(c2 // (IBLK // CHUNK))
            decode(c2, sidx0, dlb0)
            pltpu.async_copy(q_hbm.at[sidx0], rows0, sem0)

        pltpu.make_async_copy(q_hbm.at[sidx1], rows1, sem1).wait()
        process(rows1, dlb1)
        return 0

    lax.fori_loop(0, n // 2, pair, 0)

    def mrow(r, _):
        for j in range(DPW // 16):
            sl = pl.ds(j * 16, 16)
            a = plsc.bitcast(acc0[r, sl], jnp.bfloat16)
            b = plsc.bitcast(acc1[r, sl], jnp.bfloat16)
            acc0[r, sl] = plsc.bitcast(jnp.maximum(a, b), jnp.float32)
        return 0

    lax.fori_loop(0, DPT, mrow, 0)
    pltpu.sync_copy(acc0, m_hbm.at[pl.ds(lo, DPT)])


@functools.cache
def _sc_kernels():
    mesh = plsc.VectorSubcoreMesh(core_axis_name="c", subcore_axis_name="s",
                                  num_cores=2, num_subcores=16)
    params = pltpu.CompilerParams(needs_layout_passes=False)
    bucket = pl.kernel(
        _bucket_body,
        out_type=(jax.ShapeDtypeStruct((NT * CAP,), jnp.int32),
                  jax.ShapeDtypeStruct((NT * 16,), jnp.int32)),
        mesh=mesh,
        compiler_params=params,
        scratch_types=[pltpu.VMEM((EBLK,), jnp.int32),
                       pltpu.VMEM((EBLK,), jnp.int32),
                       pltpu.VMEM((EBLK,), jnp.int32),
                       pltpu.VMEM((EBLK,), jnp.int32),
                       pltpu.VMEM((EBLK + 32,), jnp.int32),
                       pltpu.VMEM((16,), jnp.int32),
                       pltpu.SemaphoreType.DMA,
                       pltpu.SemaphoreType.DMA])
    scmax = pl.kernel(
        _scmax_body,
        out_type=jax.ShapeDtypeStruct((NP, DPW), jnp.float32),
        mesh=mesh,
        compiler_params=params,
        scratch_types=[pltpu.VMEM((DPT, DPW), jnp.float32),
                       pltpu.VMEM((DPT, DPW), jnp.float32),
                       pltpu.VMEM((CHUNK, DPW), jnp.float32),
                       pltpu.VMEM((CHUNK, DPW), jnp.float32),
                       pltpu.VMEM((IBLK + 16,), jnp.int32),
                       pltpu.VMEM((CHUNK,), jnp.int32),
                       pltpu.VMEM((CHUNK,), jnp.int32),
                       pltpu.VMEM((CHUNK + 16,), jnp.int32),
                       pltpu.VMEM((CHUNK + 16,), jnp.int32),
                       pltpu.VMEM((16,), jnp.int32),
                       pltpu.SemaphoreType.DMA,
                       pltpu.SemaphoreType.DMA,
                       pltpu.SemaphoreType.DMA])
    return bucket, scmax


def kernel(x, edge_index, batch, W_i1, b_i1, W_i2, b_i2, W_p1, b_p1,
           W_p2, b_p2, W_l, b_l, W_g, b_g, W_f, b_f):
    f32 = jnp.float32
    src = edge_index[0].astype(jnp.int32)
    dst = edge_index[1].astype(jnp.int32)
    xp = jnp.pad(x.astype(f32), ((0, NP - N_NODES), (0, 0)))
    bp = jnp.pad(batch.astype(jnp.int32), (0, NP - N_NODES),
                 constant_values=N_GRAPHS)

    wl1 = W_l[:D_HID]
    wl2 = jnp.pad(W_l[D_HID:], ((0, 128 - 3), (0, 0)))
    wp2 = jnp.pad(W_p2, ((0, 0), (0, 128 - 3)))
    bp2 = jnp.pad(b_p2, (0, 128 - 3)).reshape(1, 128)

    r = lambda v: v.reshape(1, -1)
    q, pp = _tc_prep(xp, W_i1, r(b_i1), W_i2, r(b_i2), W_p1, r(b_p1),
                     wp2, bp2, wl2, wl1)
    bucket, scmax = _sc_kernels()
    plist, counts = bucket(src, dst)

    def pack(v):
        vb = v.astype(jnp.bfloat16).reshape(NP, DPW, 2)
        return jax.lax.bitcast_convert_type(vb, jnp.float32)

    def unpack(vp):
        vb = jax.lax.bitcast_convert_type(vp, jnp.bfloat16)
        return vb.reshape(NP, D_HID).astype(jnp.float32)

    h = None
    for _ in range(3):
        mp = scmax(pack(q), plist, counts)
        h, q = _tc_layer(unpack(mp), pp, W_g, r(b_g), r(b_l), wl1)
    out = _tc_pool(h, bp.reshape(GRID, 1, BM), W_f, r(b_f))
    return out


# speedup vs baseline: 1.3371x; 1.0984x over previous
import functools

import jax
import jax.numpy as jnp
from jax import lax
from jax.experimental import pallas as pl
from jax.experimental.pallas import tpu as pltpu
from jax.experimental.pallas import tpu_sc as plsc

N_NODES = 10000
N_EDGES = 320000
D_IN = 128
D_HID = 256
D_OUT = 128
N_GRAPHS = 64

NT = 32
DPT = 320
NP = NT * DPT
EBLK = 2000
NBLK_E = N_EDGES // EBLK
CHUNK = 64
CAP = N_EDGES + NBLK_E * 16 + 128
IBLK = 1024
DPW = 128
BM = 1024
GRID = NP // BM


def _prep_body(x_ref, wi1, bi1, wi2, bi2, wp1, bp1, wp2, bp2, wl2, wl1,
               q_ref, pp_ref):
    xb = x_ref[...]
    h0 = jnp.maximum(xb @ wi1[...] + bi1[...], 0.0) @ wi2[...] + bi2[...]
    p1 = jnp.maximum(h0 @ wp1[...] + bp1[...], 0.0)
    pos = p1 @ wp2[...] + bp2[...]
    pp = pos @ wl2[...]
    pp_ref[...] = pp
    q_ref[...] = h0 @ wl1[...] + pp


def _layer_body(m_ref, pp_ref, wg, bg, bl, wl1, h_ref, q_ref):
    t = m_ref[...] - pp_ref[...] + bl[...]
    h = jnp.maximum(t @ wg[...] + bg[...], 0.0)
    h_ref[...] = h
    q_ref[...] = h @ wl1[...] + pp_ref[...]


def _pool_body(h_ref, b_ref, wf, bf, out_ref, acc):
    i = pl.program_id(0)

    @pl.when(i == 0)
    def _():
        acc[...] = jnp.zeros_like(acc)

    b = b_ref[0]
    g = lax.broadcasted_iota(jnp.int32, (N_GRAPHS, BM), 0)
    oh = (b == g).astype(jnp.float32)
    acc[...] += oh @ h_ref[...]

    @pl.when(i == GRID - 1)
    def _():
        out_ref[...] = acc[...] @ wf[...] + bf[...]


def _full(a):
    return pl.BlockSpec(a.shape, lambda i: (0,) * a.ndim)


def _tc_prep(xp, wi1, bi1, wi2, bi2, wp1, bp1, wp2, bp2, wl2, wl1):
    ws = [wi1, bi1, wi2, bi2, wp1, bp1, wp2, bp2, wl2, wl1]
    return pl.pallas_call(
        _prep_body,
        grid=(GRID,),
        in_specs=[pl.BlockSpec((BM, D_IN), lambda i: (i, 0))] + [_full(w) for w in ws],
        out_specs=[pl.BlockSpec((BM, D_HID), lambda i: (i, 0))] * 2,
        out_shape=[jax.ShapeDtypeStruct((NP, D_HID), jnp.float32)] * 2,
    )(xp, *ws)


def _tc_layer(m, pp, wg, bg, bl, wl1):
    ws = [wg, bg, bl, wl1]
    return pl.pallas_call(
        _layer_body,
        grid=(GRID,),
        in_specs=[pl.BlockSpec((BM, D_HID), lambda i: (i, 0))] * 2 + [_full(w) for w in ws],
        out_specs=[pl.BlockSpec((BM, D_HID), lambda i: (i, 0))] * 2,
        out_shape=[jax.ShapeDtypeStruct((NP, D_HID), jnp.float32)] * 2,
    )(m, pp, *ws)


def _tc_pool(h, b3, wf, bf):
    return pl.pallas_call(
        _pool_body,
        grid=(GRID,),
        in_specs=[pl.BlockSpec((BM, D_HID), lambda i: (i, 0)),
                  pl.BlockSpec((1, 1, BM), lambda i: (i, 0, 0)),
                  _full(wf), _full(bf)],
        out_specs=pl.BlockSpec((N_GRAPHS, D_OUT), lambda i: (0, 0)),
        out_shape=jax.ShapeDtypeStruct((N_GRAPHS, D_OUT), jnp.float32),
        scratch_shapes=[pltpu.VMEM((N_GRAPHS, D_HID), jnp.float32)],
        compiler_params=pltpu.CompilerParams(
            dimension_semantics=("arbitrary",)),
    )(h, b3, wf, bf)


def _bucket_body(src_hbm, dst_hbm, plist, counts, sv0, dv0, sv1, dv1, ss, cs,
                 sem0, sem1):
    wid = lax.axis_index("c") * 16 + lax.axis_index("s")
    lo = wid * DPT
    hi = lo + DPT

    def issue(b, sv, dv, sem):
        pltpu.async_copy(src_hbm.at[pl.ds(b * EBLK, EBLK)], sv, sem)
        pltpu.async_copy(dst_hbm.at[pl.ds(b * EBLK, EBLK)], dv, sem)

    def wait(b, sv, dv, sem):
        pltpu.make_async_copy(src_hbm.at[pl.ds(b * EBLK, EBLK)], sv, sem).wait()
        pltpu.make_async_copy(dst_hbm.at[pl.ds(b * EBLK, EBLK)], dv, sem).wait()

    def process(sv, dv, gc):
        def ch(c, off):
            s16 = sv[pl.ds(c * 16, 16)]
            d16 = dv[pl.ds(c * 16, 16)]
            m = (d16 >= lo) & (d16 < hi)
            packed = (s16 << 9) | (d16 - lo)
            plsc.store_compressed(ss.at[pl.ds(off, 16)], packed, mask=m)
            pc = plsc.all_reduce_population_count(m)
            return off + pc[0]

        off = lax.fori_loop(0, EBLK // 16, ch, 0)
        ss[pl.ds(off, 16)] = jnp.full((16,), lo << 9, jnp.int32)
        offpad = ((off + 15) // 16) * 16
        base = pl.multiple_of(wid * CAP + gc, 16)
        pltpu.sync_copy(ss.at[pl.ds(0, EBLK + 16)],
                        plist.at[pl.ds(base, EBLK + 16)])
        return gc + offpad

    issue(0, sv0, dv0, sem0)

    def blkpair(p, gc):
        b0 = 2 * p
        issue(b0 + 1, sv1, dv1, sem1)
        wait(b0, sv0, dv0, sem0)
        gc = process(sv0, dv0, gc)

        @pl.when(b0 + 2 < NBLK_E)
        def _():
            issue(b0 + 2, sv0, dv0, sem0)

        wait(b0 + 1, sv1, dv1, sem1)
        gc = process(sv1, dv1, gc)
        return gc

    gc = lax.fori_loop(0, NBLK_E // 2, blkpair, 0)
    for k in range(128 // 16):
        ss[pl.ds(k * 16, 16)] = jnp.full((16,), lo << 9, jnp.int32)
    base = pl.multiple_of(wid * CAP + gc, 16)
    pltpu.sync_copy(ss.at[pl.ds(0, 128)], plist.at[pl.ds(base, 128)])
    padded = (gc // 128) * 128 + 128
    cs[...] = jnp.full((16,), padded, jnp.int32)
    pltpu.sync_copy(cs.at[pl.ds(0, 16)],
                    counts.at[pl.ds(pl.multiple_of(wid * 16, 16), 16)])


def _scmax_body(q_hbm, plist, counts, m_hbm, acc0, acc1, rows0, rows1, pbuf,
                sidx0, sidx1, dlb0, dlb1, cntv, sem0, sem1, sem2):
    wid = lax.axis_index("c") * 16 + lax.axis_index("s")
    lo = pl.multiple_of(wid * DPT, DPT)
    base = pl.multiple_of(wid * CAP, CHUNK)
    pltpu.sync_copy(q_hbm.at[pl.ds(lo, DPT)], acc0)
    pltpu.async_copy(q_hbm.at[pl.ds(lo, DPT)], acc1, sem2)
    pltpu.sync_copy(counts.at[pl.ds(pl.multiple_of(wid * 16, 16), 16)], cntv)
    n = cntv[pl.ds(0, 16)][0] // CHUNK

    def load_iblk(b):
        pltpu.sync_copy(plist.at[pl.ds(base + b * IBLK, IBLK)],
                        pbuf.at[pl.ds(0, IBLK)])

    def decode(c, sidx, dlb):
        o = (c % (IBLK // CHUNK)) * CHUNK
        for k in range(CHUNK // 16):
            v = pbuf[pl.ds(o + k * 16, 16)]
            sidx[pl.ds(k * 16, 16)] = v >> 9
            dlb[pl.ds(k * 16, 16)] = v & 511

    def rmw2(dl0, dl1, rows, e0):
        nj = DPW // 16

        def ld(j):
            sl = pl.ds(j * 16, 16)
            return (plsc.bitcast(acc0[dl0, sl], jnp.bfloat16),
                    plsc.bitcast(acc1[dl1, sl], jnp.bfloat16),
                    plsc.bitcast(rows[e0, sl], jnp.bfloat16),
                    plsc.bitcast(rows[e0 + 1, sl], jnp.bfloat16))

        prev = ld(0)
        for j in range(1, nj + 1):
            cur = ld(j) if j < nj else None
            a0, a1, r0, r1 = prev
            sl = pl.ds((j - 1) * 16, 16)
            acc0[dl0, sl] = plsc.bitcast(jnp.maximum(a0, r0), jnp.float32)
            acc1[dl1, sl] = plsc.bitcast(jnp.maximum(a1, r1), jnp.float32)
            prev = cur

    def process(rows, dlb):
        def quad(i, _):
            w = dlb[pl.ds(i * 4, 16)]
            rmw2(w[0], w[1], rows, i * 4)
            rmw2(w[2], w[3], rows, i * 4 + 2)
            return 0
        lax.fori_loop(0, CHUNK // 4, quad, 0)

    load_iblk(0)
    decode(0, sidx0, dlb0)
    pltpu.make_async_copy(q_hbm.at[pl.ds(lo, DPT)], acc1, sem2).wait()
    pltpu.async_copy(q_hbm.at[sidx0], rows0, sem0)

    def pair(p, _):
        c1 = 2 * p + 1
        decode(c1, sidx1, dlb1)
        pltpu.async_copy(q_hbm.at[sidx1], rows1, sem1)
        pltpu.make_async_copy(q_hbm.at[sidx0], rows0, sem0).wait()
        process(rows0, dlb0)
        c2 = 2 * p + 2

        @pl.when(c2 < n)
        def _():
            @pl.when(c2 % (IBLK // CHUNK) == 0)
            def _():
                load_iblk(c2 // (IBLK // CHUNK))
            decode(c2, sidx0, dlb0)
            pltpu.async_copy(q_hbm.at[sidx0], rows0, sem0)

        pltpu.make_async_copy(q_hbm.at[sidx1], rows1, sem1).wait()
        process(rows1, dlb1)
        return 0

    lax.fori_loop(0, n // 2, pair, 0)

    def mrow(r, _):
        for j in range(DPW // 16):
            sl = pl.ds(j * 16, 16)
            a = plsc.bitcast(acc0[r, sl], jnp.bfloat16)
            b = plsc.bitcast(acc1[r, sl], jnp.bfloat16)
            acc0[r, sl] = plsc.bitcast(jnp.maximum(a, b), jnp.float32)
        return 0

    lax.fori_loop(0, DPT, mrow, 0)
    pltpu.sync_copy(acc0, m_hbm.at[pl.ds(lo, DPT)])


@functools.cache
def _sc_kernels():
    mesh = plsc.VectorSubcoreMesh(core_axis_name="c", subcore_axis_name="s",
                                  num_cores=2, num_subcores=16)
    params = pltpu.CompilerParams(needs_layout_passes=False)
    bucket = pl.kernel(
        _bucket_body,
        out_type=(jax.ShapeDtypeStruct((NT * CAP,), jnp.int32),
                  jax.ShapeDtypeStruct((NT * 16,), jnp.int32)),
        mesh=mesh,
        compiler_params=params,
        scratch_types=[pltpu.VMEM((EBLK,), jnp.int32),
                       pltpu.VMEM((EBLK,), jnp.int32),
                       pltpu.VMEM((EBLK,), jnp.int32),
                       pltpu.VMEM((EBLK,), jnp.int32),
                       pltpu.VMEM((EBLK + 32,), jnp.int32),
                       pltpu.VMEM((16,), jnp.int32),
                       pltpu.SemaphoreType.DMA,
                       pltpu.SemaphoreType.DMA])
    scmax = pl.kernel(
        _scmax_body,
        out_type=jax.ShapeDtypeStruct((NP, DPW), jnp.float32),
        mesh=mesh,
        compiler_params=params,
        scratch_types=[pltpu.VMEM((DPT, DPW), jnp.float32),
                       pltpu.VMEM((DPT, DPW), jnp.float32),
                       pltpu.VMEM((CHUNK, DPW), jnp.float32),
                       pltpu.VMEM((CHUNK, DPW), jnp.float32),
                       pltpu.VMEM((IBLK + 16,), jnp.int32),
                       pltpu.VMEM((CHUNK,), jnp.int32),
                       pltpu.VMEM((CHUNK,), jnp.int32),
                       pltpu.VMEM((CHUNK + 16,), jnp.int32),
                       pltpu.VMEM((CHUNK + 16,), jnp.int32),
                       pltpu.VMEM((16,), jnp.int32),
                       pltpu.SemaphoreType.DMA,
                       pltpu.SemaphoreType.DMA,
                       pltpu.SemaphoreType.DMA])
    return bucket, scmax


def kernel(x, edge_index, batch, W_i1, b_i1, W_i2, b_i2, W_p1, b_p1,
           W_p2, b_p2, W_l, b_l, W_g, b_g, W_f, b_f):
    f32 = jnp.float32
    src = edge_index[0].astype(jnp.int32)
    dst = edge_index[1].astype(jnp.int32)
    xp = jnp.pad(x.astype(f32), ((0, NP - N_NODES), (0, 0)))
    bp = jnp.pad(batch.astype(jnp.int32), (0, NP - N_NODES),
                 constant_values=N_GRAPHS)

    wl1 = W_l[:D_HID]
    wl2 = jnp.pad(W_l[D_HID:], ((0, 128 - 3), (0, 0)))
    wp2 = jnp.pad(W_p2, ((0, 0), (0, 128 - 3)))
    bp2 = jnp.pad(b_p2, (0, 128 - 3)).reshape(1, 128)

    r = lambda v: v.reshape(1, -1)
    q, pp = _tc_prep(xp, W_i1, r(b_i1), W_i2, r(b_i2), W_p1, r(b_p1),
                     wp2, bp2, wl2, wl1)
    bucket, scmax = _sc_kernels()
    plist, counts = bucket(src, dst)

    def pack(v):
        vb = v.astype(jnp.bfloat16).reshape(NP, DPW, 2)
        return jax.lax.bitcast_convert_type(vb, jnp.float32)

    def unpack(vp):
        vb = jax.lax.bitcast_convert_type(vp, jnp.bfloat16)
        return vb.reshape(NP, D_HID).astype(jnp.float32)

    h = None
    for _ in range(3):
        mp = scmax(pack(q), plist, counts)
        h, q = _tc_layer(unpack(mp), pp, W_g, r(b_g), r(b_l), wl1)
    out = _tc_pool(h, bp.reshape(GRID, 1, BM), W_f, r(b_f))
    return out


# speedup vs baseline: 1.3861x; 1.0367x over previous
import functools

import jax
import jax.numpy as jnp
from jax import lax
from jax.experimental import pallas as pl
from jax.experimental.pallas import tpu as pltpu
from jax.experimental.pallas import tpu_sc as plsc

N_NODES = 10000
N_EDGES = 320000
D_IN = 128
D_HID = 256
D_OUT = 128
N_GRAPHS = 64

NT = 32
DPT = 320
NP = NT * DPT
EBLK = 2000
NBLK_E = N_EDGES // EBLK
CHUNK = 128
CAP = N_EDGES + NBLK_E * 16 + 256
IBLK = 1024
DPW = 128
BM = 1024
GRID = NP // BM


def _prep_body(x_ref, wi1, bi1, wi2, bi2, wp1, bp1, wp2, bp2, wl2, wl1,
               q_ref, pp_ref):
    xb = x_ref[...]
    h0 = jnp.maximum(xb @ wi1[...] + bi1[...], 0.0) @ wi2[...] + bi2[...]
    p1 = jnp.maximum(h0 @ wp1[...] + bp1[...], 0.0)
    pos = p1 @ wp2[...] + bp2[...]
    pp = pos @ wl2[...]
    pp_ref[...] = pp
    q_ref[...] = h0 @ wl1[...] + pp


def _layer_body(m_ref, pp_ref, wg, bg, bl, wl1, h_ref, q_ref):
    t = m_ref[...] - pp_ref[...] + bl[...]
    h = jnp.maximum(t @ wg[...] + bg[...], 0.0)
    h_ref[...] = h
    q_ref[...] = h @ wl1[...] + pp_ref[...]


def _pool_body(h_ref, b_ref, wf, bf, out_ref, acc):
    i = pl.program_id(0)

    @pl.when(i == 0)
    def _():
        acc[...] = jnp.zeros_like(acc)

    b = b_ref[0]
    g = lax.broadcasted_iota(jnp.int32, (N_GRAPHS, BM), 0)
    oh = (b == g).astype(jnp.float32)
    acc[...] += oh @ h_ref[...]

    @pl.when(i == GRID - 1)
    def _():
        out_ref[...] = acc[...] @ wf[...] + bf[...]


def _full(a):
    return pl.BlockSpec(a.shape, lambda i: (0,) * a.ndim)


def _tc_prep(xp, wi1, bi1, wi2, bi2, wp1, bp1, wp2, bp2, wl2, wl1):
    ws = [wi1, bi1, wi2, bi2, wp1, bp1, wp2, bp2, wl2, wl1]
    return pl.pallas_call(
        _prep_body,
        grid=(GRID,),
        in_specs=[pl.BlockSpec((BM, D_IN), lambda i: (i, 0))] + [_full(w) for w in ws],
        out_specs=[pl.BlockSpec((BM, D_HID), lambda i: (i, 0))] * 2,
        out_shape=[jax.ShapeDtypeStruct((NP, D_HID), jnp.float32)] * 2,
    )(xp, *ws)


def _tc_layer(m, pp, wg, bg, bl, wl1):
    ws = [wg, bg, bl, wl1]
    return pl.pallas_call(
        _layer_body,
        grid=(GRID,),
        in_specs=[pl.BlockSpec((BM, D_HID), lambda i: (i, 0))] * 2 + [_full(w) for w in ws],
        out_specs=[pl.BlockSpec((BM, D_HID), lambda i: (i, 0))] * 2,
        out_shape=[jax.ShapeDtypeStruct((NP, D_HID), jnp.float32)] * 2,
    )(m, pp, *ws)


def _tc_pool(h, b3, wf, bf):
    return pl.pallas_call(
        _pool_body,
        grid=(GRID,),
        in_specs=[pl.BlockSpec((BM, D_HID), lambda i: (i, 0)),
                  pl.BlockSpec((1, 1, BM), lambda i: (i, 0, 0)),
                  _full(wf), _full(bf)],
        out_specs=pl.BlockSpec((N_GRAPHS, D_OUT), lambda i: (0, 0)),
        out_shape=jax.ShapeDtypeStruct((N_GRAPHS, D_OUT), jnp.float32),
        scratch_shapes=[pltpu.VMEM((N_GRAPHS, D_HID), jnp.float32)],
        compiler_params=pltpu.CompilerParams(
            dimension_semantics=("arbitrary",)),
    )(h, b3, wf, bf)


def _bucket_body(src_hbm, dst_hbm, plist, counts, sv0, dv0, sv1, dv1, ss, cs,
                 sem0, sem1):
    wid = lax.axis_index("c") * 16 + lax.axis_index("s")
    lo = wid * DPT
    hi = lo + DPT

    def issue(b, sv, dv, sem):
        pltpu.async_copy(src_hbm.at[pl.ds(b * EBLK, EBLK)], sv, sem)
        pltpu.async_copy(dst_hbm.at[pl.ds(b * EBLK, EBLK)], dv, sem)

    def wait(b, sv, dv, sem):
        pltpu.make_async_copy(src_hbm.at[pl.ds(b * EBLK, EBLK)], sv, sem).wait()
        pltpu.make_async_copy(dst_hbm.at[pl.ds(b * EBLK, EBLK)], dv, sem).wait()

    def process(sv, dv, gc):
        def ch(c, off):
            s16 = sv[pl.ds(c * 16, 16)]
            d16 = dv[pl.ds(c * 16, 16)]
            m = (d16 >= lo) & (d16 < hi)
            packed = (s16 << 9) | (d16 - lo)
            plsc.store_compressed(ss.at[pl.ds(off, 16)], packed, mask=m)
            pc = plsc.all_reduce_population_count(m)
            return off + pc[0]

        off = lax.fori_loop(0, EBLK // 16, ch, 0)
        ss[pl.ds(off, 16)] = jnp.full((16,), lo << 9, jnp.int32)
        offpad = ((off + 15) // 16) * 16
        base = pl.multiple_of(wid * CAP + gc, 16)
        pltpu.sync_copy(ss.at[pl.ds(0, EBLK + 16)],
                        plist.at[pl.ds(base, EBLK + 16)])
        return gc + offpad

    issue(0, sv0, dv0, sem0)

    def blkpair(p, gc):
        b0 = 2 * p
        issue(b0 + 1, sv1, dv1, sem1)
        wait(b0, sv0, dv0, sem0)
        gc = process(sv0, dv0, gc)

        @pl.when(b0 + 2 < NBLK_E)
        def _():
            issue(b0 + 2, sv0, dv0, sem0)

        wait(b0 + 1, sv1, dv1, sem1)
        gc = process(sv1, dv1, gc)
        return gc

    gc = lax.fori_loop(0, NBLK_E // 2, blkpair, 0)
    for k in range(256 // 16):
        ss[pl.ds(k * 16, 16)] = jnp.full((16,), lo << 9, jnp.int32)
    base = pl.multiple_of(wid * CAP + gc, 16)
    pltpu.sync_copy(ss.at[pl.ds(0, 256)], plist.at[pl.ds(base, 256)])
    padded = (gc // 256) * 256 + 256
    cs[...] = jnp.full((16,), padded, jnp.int32)
    pltpu.sync_copy(cs.at[pl.ds(0, 16)],
                    counts.at[pl.ds(pl.multiple_of(wid * 16, 16), 16)])


def _scmax_body(q_hbm, plist, counts, m_hbm, acc0, acc1, rows0, rows1, pbuf,
                sidx0, sidx1, dlb0, dlb1, cntv, sem0, sem1, sem2):
    wid = lax.axis_index("c") * 16 + lax.axis_index("s")
    lo = pl.multiple_of(wid * DPT, DPT)
    base = pl.multiple_of(wid * CAP, CHUNK)
    pltpu.sync_copy(q_hbm.at[pl.ds(lo, DPT)], acc0)
    pltpu.async_copy(q_hbm.at[pl.ds(lo, DPT)], acc1, sem2)
    pltpu.sync_copy(counts.at[pl.ds(pl.multiple_of(wid * 16, 16), 16)], cntv)
    n = cntv[pl.ds(0, 16)][0] // CHUNK

    def load_iblk(b):
        pltpu.sync_copy(plist.at[pl.ds(base + b * IBLK, IBLK)],
                        pbuf.at[pl.ds(0, IBLK)])

    def decode(c, sidx, dlb):
        o = (c % (IBLK // CHUNK)) * CHUNK
        for k in range(CHUNK // 16):
            v = pbuf[pl.ds(o + k * 16, 16)]
            sidx[pl.ds(k * 16, 16)] = v >> 9
            dlb[pl.ds(k * 16, 16)] = v & 511

    def rmw2(dl0, dl1, rows, e0):
        nj = DPW // 16

        def ld(j):
            sl = pl.ds(j * 16, 16)
            return (plsc.bitcast(acc0[dl0, sl], jnp.bfloat16),
                    plsc.bitcast(acc1[dl1, sl], jnp.bfloat16),
                    plsc.bitcast(rows[e0, sl], jnp.bfloat16),
                    plsc.bitcast(rows[e0 + 1, sl], jnp.bfloat16))

        prev = ld(0)
        for j in range(1, nj + 1):
            cur = ld(j) if j < nj else None
            a0, a1, r0, r1 = prev
            sl = pl.ds((j - 1) * 16, 16)
            acc0[dl0, sl] = plsc.bitcast(jnp.maximum(a0, r0), jnp.float32)
            acc1[dl1, sl] = plsc.bitcast(jnp.maximum(a1, r1), jnp.float32)
            prev = cur

    def process(rows, dlb):
        def quad(i, _):
            w = dlb[pl.ds(i * 4, 16)]
            rmw2(w[0], w[1], rows, i * 4)
            rmw2(w[2], w[3], rows, i * 4 + 2)
            return 0
        lax.fori_loop(0, CHUNK // 4, quad, 0)

    load_iblk(0)
    decode(0, sidx0, dlb0)
    pltpu.make_async_copy(q_hbm.at[pl.ds(lo, DPT)], acc1, sem2).wait()
    pltpu.async_copy(q_hbm.at[sidx0], rows0, sem0)

    def pair(p, _):
        c1 = 2 * p + 1
        decode(c1, sidx1, dlb1)
        pltpu.async_copy(q_hbm.at[sidx1], rows1, sem1)
        pltpu.make_async_copy(q_hbm.at[sidx0], rows0, sem0).wait()
        process(rows0, dlb0)
        c2 = 2 * p + 2

        @pl.when(c2 < n)
        def _():
            @pl.when(c2 % (IBLK // CHUNK) == 0)
            def _():
                load_iblk(c2 // (IBLK // CHUNK))
            decode(c2, sidx0, dlb0)
            pltpu.async_copy(q_hbm.at[sidx0], rows0, sem0)

        pltpu.make_async_copy(q_hbm.at[sidx1], rows1, sem1).wait()
        process(rows1, dlb1)
        return 0

    lax.fori_loop(0, n // 2, pair, 0)

    def mrow(r, _):
        for j in range(DPW // 16):
            sl = pl.ds(j * 16, 16)
            a = plsc.bitcast(acc0[r, sl], jnp.bfloat16)
            b = plsc.bitcast(acc1[r, sl], jnp.bfloat16)
            acc0[r, sl] = plsc.bitcast(jnp.maximum(a, b), jnp.float32)
        return 0

    lax.fori_loop(0, DPT, mrow, 0)
    pltpu.sync_copy(acc0, m_hbm.at[pl.ds(lo, DPT)])


@functools.cache
def _sc_kernels():
    mesh = plsc.VectorSubcoreMesh(core_axis_name="c", subcore_axis_name="s",
                                  num_cores=2, num_subcores=16)
    params = pltpu.CompilerParams(needs_layout_passes=False)
    bucket = pl.kernel(
        _bucket_body,
        out_type=(jax.ShapeDtypeStruct((NT * CAP,), jnp.int32),
                  jax.ShapeDtypeStruct((NT * 16,), jnp.int32)),
        mesh=mesh,
        compiler_params=params,
        scratch_types=[pltpu.VMEM((EBLK,), jnp.int32),
                       pltpu.VMEM((EBLK,), jnp.int32),
                       pltpu.VMEM((EBLK,), jnp.int32),
                       pltpu.VMEM((EBLK,), jnp.int32),
                       pltpu.VMEM((EBLK + 32,), jnp.int32),
                       pltpu.VMEM((16,), jnp.int32),
                       pltpu.SemaphoreType.DMA,
                       pltpu.SemaphoreType.DMA])
    scmax = pl.kernel(
        _scmax_body,
        out_type=jax.ShapeDtypeStruct((NP, DPW), jnp.float32),
        mesh=mesh,
        compiler_params=params,
        scratch_types=[pltpu.VMEM((DPT, DPW), jnp.float32),
                       pltpu.VMEM((DPT, DPW), jnp.float32),
                       pltpu.VMEM((CHUNK, DPW), jnp.float32),
                       pltpu.VMEM((CHUNK, DPW), jnp.float32),
                       pltpu.VMEM((IBLK + 16,), jnp.int32),
                       pltpu.VMEM((CHUNK,), jnp.int32),
                       pltpu.VMEM((CHUNK,), jnp.int32),
                       pltpu.VMEM((CHUNK + 16,), jnp.int32),
                       pltpu.VMEM((CHUNK + 16,), jnp.int32),
                       pltpu.VMEM((16,), jnp.int32),
                       pltpu.SemaphoreType.DMA,
                       pltpu.SemaphoreType.DMA,
                       pltpu.SemaphoreType.DMA])
    return bucket, scmax


def kernel(x, edge_index, batch, W_i1, b_i1, W_i2, b_i2, W_p1, b_p1,
           W_p2, b_p2, W_l, b_l, W_g, b_g, W_f, b_f):
    f32 = jnp.float32
    src = edge_index[0].astype(jnp.int32)
    dst = edge_index[1].astype(jnp.int32)
    xp = jnp.pad(x.astype(f32), ((0, NP - N_NODES), (0, 0)))
    bp = jnp.pad(batch.astype(jnp.int32), (0, NP - N_NODES),
                 constant_values=N_GRAPHS)

    wl1 = W_l[:D_HID]
    wl2 = jnp.pad(W_l[D_HID:], ((0, 128 - 3), (0, 0)))
    wp2 = jnp.pad(W_p2, ((0, 0), (0, 128 - 3)))
    bp2 = jnp.pad(b_p2, (0, 128 - 3)).reshape(1, 128)

    r = lambda v: v.reshape(1, -1)
    q, pp = _tc_prep(xp, W_i1, r(b_i1), W_i2, r(b_i2), W_p1, r(b_p1),
                     wp2, bp2, wl2, wl1)
    bucket, scmax = _sc_kernels()
    plist, counts = bucket(src, dst)

    def pack(v):
        vb = v.astype(jnp.bfloat16).reshape(NP, DPW, 2)
        return jax.lax.bitcast_convert_type(vb, jnp.float32)

    def unpack(vp):
        vb = jax.lax.bitcast_convert_type(vp, jnp.bfloat16)
        return vb.reshape(NP, D_HID).astype(jnp.float32)

    h = None
    for _ in range(3):
        mp = scmax(pack(q), plist, counts)
        h, q = _tc_layer(unpack(mp), pp, W_g, r(b_g), r(b_l), wl1)
    out = _tc_pool(h, bp.reshape(GRID, 1, BM), W_f, r(b_f))
    return out


# speedup vs baseline: 1.9133x; 1.3804x over previous
import functools

import jax
import jax.numpy as jnp
from jax import lax
from jax.experimental import pallas as pl
from jax.experimental.pallas import tpu as pltpu
from jax.experimental.pallas import tpu_sc as plsc

N_NODES = 10000
N_EDGES = 320000
D_IN = 128
D_HID = 256
D_OUT = 128
N_GRAPHS = 64

NT = 32
DPT = 320
NP = NT * DPT
EBLK = 2000
NBLK_E = N_EDGES // EBLK
CHUNK = 128
CAP = N_EDGES + NBLK_E * 16 + 256
IBLK = 1024
DPW = 128
BM = 1024
GRID = NP // BM


def _pack_tc(x):
    lob = jax.lax.bitcast_convert_type(
        x[:, :DPW].astype(jnp.bfloat16), jnp.uint16).astype(jnp.uint32)
    hib = jax.lax.bitcast_convert_type(
        x[:, DPW:].astype(jnp.bfloat16), jnp.uint16).astype(jnp.uint32)
    return jax.lax.bitcast_convert_type(lob | (hib << 16), jnp.float32)


def _unpack_tc(xp):
    u = jax.lax.bitcast_convert_type(xp, jnp.uint32)
    xlo = jax.lax.bitcast_convert_type(u << 16, jnp.float32)
    xhi = jax.lax.bitcast_convert_type(u & jnp.uint32(0xFFFF0000), jnp.float32)
    return jnp.concatenate([xlo, xhi], axis=1)


def _prep_body(x_ref, wi1, bi1, wi2, bi2, wp1, bp1, wp2, bp2, wl2, wl1,
               qp_ref, pp_ref):
    xb = x_ref[...]
    h0 = jnp.maximum(xb @ wi1[...] + bi1[...], 0.0) @ wi2[...] + bi2[...]
    p1 = jnp.maximum(h0 @ wp1[...] + bp1[...], 0.0)
    pos = p1 @ wp2[...] + bp2[...]
    pp = pos @ wl2[...]
    pp_ref[...] = pp
    qp_ref[...] = _pack_tc(h0 @ wl1[...] + pp)


def _layer_body(mp_ref, pp_ref, wg, bg, bl, wl1, h_ref, qp_ref):
    t = _unpack_tc(mp_ref[...]) - pp_ref[...] + bl[...]
    h = jnp.maximum(t @ wg[...] + bg[...], 0.0)
    h_ref[...] = h
    qp_ref[...] = _pack_tc(h @ wl1[...] + pp_ref[...])


def _pool_body(h_ref, b_ref, wf, bf, out_ref, acc):
    i = pl.program_id(0)

    @pl.when(i == 0)
    def _():
        acc[...] = jnp.zeros_like(acc)

    b = b_ref[0]
    g = lax.broadcasted_iota(jnp.int32, (N_GRAPHS, BM), 0)
    oh = (b == g).astype(jnp.float32)
    acc[...] += oh @ h_ref[...]

    @pl.when(i == GRID - 1)
    def _():
        out_ref[...] = acc[...] @ wf[...] + bf[...]


def _full(a):
    return pl.BlockSpec(a.shape, lambda i: (0,) * a.ndim)


def _tc_prep(xp, wi1, bi1, wi2, bi2, wp1, bp1, wp2, bp2, wl2, wl1):
    ws = [wi1, bi1, wi2, bi2, wp1, bp1, wp2, bp2, wl2, wl1]
    return pl.pallas_call(
        _prep_body,
        grid=(GRID,),
        in_specs=[pl.BlockSpec((BM, D_IN), lambda i: (i, 0))] + [_full(w) for w in ws],
        out_specs=[pl.BlockSpec((BM, DPW), lambda i: (i, 0)),
                   pl.BlockSpec((BM, D_HID), lambda i: (i, 0))],
        out_shape=[jax.ShapeDtypeStruct((NP, DPW), jnp.float32),
                   jax.ShapeDtypeStruct((NP, D_HID), jnp.float32)],
    )(xp, *ws)


def _tc_layer(mp, pp, wg, bg, bl, wl1):
    ws = [wg, bg, bl, wl1]
    return pl.pallas_call(
        _layer_body,
        grid=(GRID,),
        in_specs=[pl.BlockSpec((BM, DPW), lambda i: (i, 0)),
                  pl.BlockSpec((BM, D_HID), lambda i: (i, 0))] + [_full(w) for w in ws],
        out_specs=[pl.BlockSpec((BM, D_HID), lambda i: (i, 0)),
                   pl.BlockSpec((BM, DPW), lambda i: (i, 0))],
        out_shape=[jax.ShapeDtypeStruct((NP, D_HID), jnp.float32),
                   jax.ShapeDtypeStruct((NP, DPW), jnp.float32)],
    )(mp, pp, *ws)


def _tc_pool(h, b3, wf, bf):
    return pl.pallas_call(
        _pool_body,
        grid=(GRID,),
        in_specs=[pl.BlockSpec((BM, D_HID), lambda i: (i, 0)),
                  pl.BlockSpec((1, 1, BM), lambda i: (i, 0, 0)),
                  _full(wf), _full(bf)],
        out_specs=pl.BlockSpec((N_GRAPHS, D_OUT), lambda i: (0, 0)),
        out_shape=jax.ShapeDtypeStruct((N_GRAPHS, D_OUT), jnp.float32),
        scratch_shapes=[pltpu.VMEM((N_GRAPHS, D_HID), jnp.float32)],
        compiler_params=pltpu.CompilerParams(
            dimension_semantics=("arbitrary",)),
    )(h, b3, wf, bf)


def _bucket_body(src_hbm, dst_hbm, plist, counts, sv0, dv0, sv1, dv1, ss, cs,
                 sem0, sem1):
    wid = lax.axis_index("c") * 16 + lax.axis_index("s")
    lo = wid * DPT
    hi = lo + DPT

    def issue(b, sv, dv, sem):
        pltpu.async_copy(src_hbm.at[pl.ds(b * EBLK, EBLK)], sv, sem)
        pltpu.async_copy(dst_hbm.at[pl.ds(b * EBLK, EBLK)], dv, sem)

    def wait(b, sv, dv, sem):
        pltpu.make_async_copy(src_hbm.at[pl.ds(b * EBLK, EBLK)], sv, sem).wait()
        pltpu.make_async_copy(dst_hbm.at[pl.ds(b * EBLK, EBLK)], dv, sem).wait()

    def process(sv, dv, gc):
        def ch(c, off):
            s16 = sv[pl.ds(c * 16, 16)]
            d16 = dv[pl.ds(c * 16, 16)]
            m = (d16 >= lo) & (d16 < hi)
            packed = (s16 << 9) | (d16 - lo)
            plsc.store_compressed(ss.at[pl.ds(off, 16)], packed, mask=m)
            pc = plsc.all_reduce_population_count(m)
            return off + pc[0]

        off = lax.fori_loop(0, EBLK // 16, ch, 0)
        ss[pl.ds(off, 16)] = jnp.full((16,), lo << 9, jnp.int32)
        offpad = ((off + 15) // 16) * 16
        base = pl.multiple_of(wid * CAP + gc, 16)
        pltpu.sync_copy(ss.at[pl.ds(0, EBLK + 16)],
                        plist.at[pl.ds(base, EBLK + 16)])
        return gc + offpad

    issue(0, sv0, dv0, sem0)

    def blkpair(p, gc):
        b0 = 2 * p
        issue(b0 + 1, sv1, dv1, sem1)
        wait(b0, sv0, dv0, sem0)
        gc = process(sv0, dv0, gc)

        @pl.when(b0 + 2 < NBLK_E)
        def _():
            issue(b0 + 2, sv0, dv0, sem0)

        wait(b0 + 1, sv1, dv1, sem1)
        gc = process(sv1, dv1, gc)
        return gc

    gc = lax.fori_loop(0, NBLK_E // 2, blkpair, 0)
    for k in range(256 // 16):
        ss[pl.ds(k * 16, 16)] = jnp.full((16,), lo << 9, jnp.int32)
    base = pl.multiple_of(wid * CAP + gc, 16)
    pltpu.sync_copy(ss.at[pl.ds(0, 256)], plist.at[pl.ds(base, 256)])
    padded = (gc // 256) * 256 + 256
    cs[...] = jnp.full((16,), padded, jnp.int32)
    pltpu.sync_copy(cs.at[pl.ds(0, 16)],
                    counts.at[pl.ds(pl.multiple_of(wid * 16, 16), 16)])


def _scmax_body(q_hbm, plist, counts, m_hbm, acc0, acc1, rows0, rows1, pbuf,
                sidx0, sidx1, dlb0, dlb1, cntv, sem0, sem1, sem2):
    wid = lax.axis_index("c") * 16 + lax.axis_index("s")
    lo = pl.multiple_of(wid * DPT, DPT)
    base = pl.multiple_of(wid * CAP, CHUNK)
    pltpu.sync_copy(q_hbm.at[pl.ds(lo, DPT)], acc0)
    pltpu.async_copy(q_hbm.at[pl.ds(lo, DPT)], acc1, sem2)
    pltpu.sync_copy(counts.at[pl.ds(pl.multiple_of(wid * 16, 16), 16)], cntv)
    n = cntv[pl.ds(0, 16)][0] // CHUNK

    def load_iblk(b):
        pltpu.sync_copy(plist.at[pl.ds(base + b * IBLK, IBLK)],
                        pbuf.at[pl.ds(0, IBLK)])

    def decode(c, sidx, dlb):
        o = (c % (IBLK // CHUNK)) * CHUNK
        for k in range(CHUNK // 16):
            v = pbuf[pl.ds(o + k * 16, 16)]
            sidx[pl.ds(k * 16, 16)] = v >> 9
            dlb[pl.ds(k * 16, 16)] = v & 511

    def rmw2(dl0, dl1, rows, e0):
        nj = DPW // 16

        def ld(j):
            sl = pl.ds(j * 16, 16)
            return (plsc.bitcast(acc0[dl0, sl], jnp.bfloat16),
                    plsc.bitcast(acc1[dl1, sl], jnp.bfloat16),
                    plsc.bitcast(rows[e0, sl], jnp.bfloat16),
                    plsc.bitcast(rows[e0 + 1, sl], jnp.bfloat16))

        prev = ld(0)
        for j in range(1, nj + 1):
            cur = ld(j) if j < nj else None
            a0, a1, r0, r1 = prev
            sl = pl.ds((j - 1) * 16, 16)
            acc0[dl0, sl] = plsc.bitcast(jnp.maximum(a0, r0), jnp.float32)
            acc1[dl1, sl] = plsc.bitcast(jnp.maximum(a1, r1), jnp.float32)
            prev = cur

    def process(rows, dlb):
        def quad(i, _):
            w = dlb[pl.ds(i * 4, 16)]
            rmw2(w[0], w[1], rows, i * 4)
            rmw2(w[2], w[3], rows, i * 4 + 2)
            return 0
        lax.fori_loop(0, CHUNK // 4, quad, 0)

    load_iblk(0)
    decode(0, sidx0, dlb0)
    pltpu.make_async_copy(q_hbm.at[pl.ds(lo, DPT)], acc1, sem2).wait()
    pltpu.async_copy(q_hbm.at[sidx0], rows0, sem0)

    def pair(p, _):
        c1 = 2 * p + 1
        decode(c1, sidx1, dlb1)
        pltpu.async_copy(q_hbm.at[sidx1], rows1, sem1)
        pltpu.make_async_copy(q_hbm.at[sidx0], rows0, sem0).wait()
        process(rows0, dlb0)
        c2 = 2 * p + 2

        @pl.when(c2 < n)
        def _():
            @pl.when(c2 % (IBLK // CHUNK) == 0)
            def _():
                load_iblk(c2 // (IBLK // CHUNK))
            decode(c2, sidx0, dlb0)
            pltpu.async_copy(q_hbm.at[sidx0], rows0, sem0)

        pltpu.make_async_copy(q_hbm.at[sidx1], rows1, sem1).wait()
        process(rows1, dlb1)
        return 0

    lax.fori_loop(0, n // 2, pair, 0)

    def mrow(r, _):
        for j in range(DPW // 16):
            sl = pl.ds(j * 16, 16)
            a = plsc.bitcast(acc0[r, sl], jnp.bfloat16)
            b = plsc.bitcast(acc1[r, sl], jnp.bfloat16)
            acc0[r, sl] = plsc.bitcast(jnp.maximum(a, b), jnp.float32)
        return 0

    lax.fori_loop(0, DPT, mrow, 0)
    pltpu.sync_copy(acc0, m_hbm.at[pl.ds(lo, DPT)])


@functools.cache
def _sc_kernels():
    mesh = plsc.VectorSubcoreMesh(core_axis_name="c", subcore_axis_name="s",
                                  num_cores=2, num_subcores=16)
    params = pltpu.CompilerParams(needs_layout_passes=False)
    bucket = pl.kernel(
        _bucket_body,
        out_type=(jax.ShapeDtypeStruct((NT * CAP,), jnp.int32),
                  jax.ShapeDtypeStruct((NT * 16,), jnp.int32)),
        mesh=mesh,
        compiler_params=params,
        scratch_types=[pltpu.VMEM((EBLK,), jnp.int32),
                       pltpu.VMEM((EBLK,), jnp.int32),
                       pltpu.VMEM((EBLK,), jnp.int32),
                       pltpu.VMEM((EBLK,), jnp.int32),
                       pltpu.VMEM((EBLK + 32,), jnp.int32),
                       pltpu.VMEM((16,), jnp.int32),
                       pltpu.SemaphoreType.DMA,
                       pltpu.SemaphoreType.DMA])
    scmax = pl.kernel(
        _scmax_body,
        out_type=jax.ShapeDtypeStruct((NP, DPW), jnp.float32),
        mesh=mesh,
        compiler_params=params,
        scratch_types=[pltpu.VMEM((DPT, DPW), jnp.float32),
                       pltpu.VMEM((DPT, DPW), jnp.float32),
                       pltpu.VMEM((CHUNK, DPW), jnp.float32),
                       pltpu.VMEM((CHUNK, DPW), jnp.float32),
                       pltpu.VMEM((IBLK + 16,), jnp.int32),
                       pltpu.VMEM((CHUNK,), jnp.int32),
                       pltpu.VMEM((CHUNK,), jnp.int32),
                       pltpu.VMEM((CHUNK + 16,), jnp.int32),
                       pltpu.VMEM((CHUNK + 16,), jnp.int32),
                       pltpu.VMEM((16,), jnp.int32),
                       pltpu.SemaphoreType.DMA,
                       pltpu.SemaphoreType.DMA,
                       pltpu.SemaphoreType.DMA])
    return bucket, scmax


def kernel(x, edge_index, batch, W_i1, b_i1, W_i2, b_i2, W_p1, b_p1,
           W_p2, b_p2, W_l, b_l, W_g, b_g, W_f, b_f):
    f32 = jnp.float32
    src = edge_index[0].astype(jnp.int32)
    dst = edge_index[1].astype(jnp.int32)
    xp = jnp.pad(x.astype(f32), ((0, NP - N_NODES), (0, 0)))
    bp = jnp.pad(batch.astype(jnp.int32), (0, NP - N_NODES),
                 constant_values=N_GRAPHS)

    wl1 = W_l[:D_HID]
    wl2 = jnp.pad(W_l[D_HID:], ((0, 128 - 3), (0, 0)))
    wp2 = jnp.pad(W_p2, ((0, 0), (0, 128 - 3)))
    bp2 = jnp.pad(b_p2, (0, 128 - 3)).reshape(1, 128)

    r = lambda v: v.reshape(1, -1)
    qp, pp = _tc_prep(xp, W_i1, r(b_i1), W_i2, r(b_i2), W_p1, r(b_p1),
                     wp2, bp2, wl2, wl1)
    bucket, scmax = _sc_kernels()
    plist, counts = bucket(src, dst)
    h = None
    for _ in range(3):
        mp = scmax(qp, plist, counts)
        h, qp = _tc_layer(mp, pp, W_g, r(b_g), r(b_l), wl1)
    out = _tc_pool(h, bp.reshape(GRID, 1, BM), W_f, r(b_f))
    return out


# speedup vs baseline: 2.2326x; 1.1669x over previous
import functools

import jax
import jax.numpy as jnp
from jax import lax
from jax.experimental import pallas as pl
from jax.experimental.pallas import tpu as pltpu
from jax.experimental.pallas import tpu_sc as plsc

N_NODES = 10000
N_EDGES = 320000
D_IN = 128
D_HID = 256
D_OUT = 128
N_GRAPHS = 64

NT = 32
DPT = 320
NP = NT * DPT
EBLK = 4000
NBLK_E = N_EDGES // EBLK
CHUNK = 128
CAP = N_EDGES + NBLK_E * 16 + 256
IBLK = 1024
DPW = 128
BM = 1024
GRID = NP // BM


def _pack_tc(x):
    lob = jax.lax.bitcast_convert_type(
        x[:, :DPW].astype(jnp.bfloat16), jnp.uint16).astype(jnp.uint32)
    hib = jax.lax.bitcast_convert_type(
        x[:, DPW:].astype(jnp.bfloat16), jnp.uint16).astype(jnp.uint32)
    return jax.lax.bitcast_convert_type(lob | (hib << 16), jnp.float32)


def _unpack_tc(xp):
    u = jax.lax.bitcast_convert_type(xp, jnp.uint32)
    xlo = jax.lax.bitcast_convert_type(u << 16, jnp.float32)
    xhi = jax.lax.bitcast_convert_type(u & jnp.uint32(0xFFFF0000), jnp.float32)
    return jnp.concatenate([xlo, xhi], axis=1)


def _prep_body(x_ref, wi1, bi1, wi2, bi2, wp1, bp1, wp2, bp2, wl2, wl1,
               qp_ref, pp_ref):
    xb = x_ref[...]
    h0 = jnp.maximum(xb @ wi1[...] + bi1[...], 0.0) @ wi2[...] + bi2[...]
    p1 = jnp.maximum(h0 @ wp1[...] + bp1[...], 0.0)
    pos = p1 @ wp2[...] + bp2[...]
    pp = pos @ wl2[...]
    pp_ref[...] = pp
    qp_ref[...] = _pack_tc(h0 @ wl1[...] + pp)


def _layer_body(mp_ref, pp_ref, wg, bg, bl, wl1, h_ref, qp_ref):
    t = _unpack_tc(mp_ref[...]) - pp_ref[...] + bl[...]
    h = jnp.maximum(t @ wg[...] + bg[...], 0.0)
    h_ref[...] = h
    qp_ref[...] = _pack_tc(h @ wl1[...] + pp_ref[...])


def _pool_body(h_ref, b_ref, wf, bf, out_ref, acc):
    i = pl.program_id(0)

    @pl.when(i == 0)
    def _():
        acc[...] = jnp.zeros_like(acc)

    b = b_ref[0]
    g = lax.broadcasted_iota(jnp.int32, (N_GRAPHS, BM), 0)
    oh = (b == g).astype(jnp.float32)
    acc[...] += oh @ h_ref[...]

    @pl.when(i == GRID - 1)
    def _():
        out_ref[...] = acc[...] @ wf[...] + bf[...]


def _full(a):
    return pl.BlockSpec(a.shape, lambda i: (0,) * a.ndim)


def _tc_prep(xp, wi1, bi1, wi2, bi2, wp1, bp1, wp2, bp2, wl2, wl1):
    ws = [wi1, bi1, wi2, bi2, wp1, bp1, wp2, bp2, wl2, wl1]
    return pl.pallas_call(
        _prep_body,
        grid=(GRID,),
        in_specs=[pl.BlockSpec((BM, D_IN), lambda i: (i, 0))] + [_full(w) for w in ws],
        out_specs=[pl.BlockSpec((BM, DPW), lambda i: (i, 0)),
                   pl.BlockSpec((BM, D_HID), lambda i: (i, 0))],
        out_shape=[jax.ShapeDtypeStruct((NP, DPW), jnp.float32),
                   jax.ShapeDtypeStruct((NP, D_HID), jnp.float32)],
    )(xp, *ws)


def _tc_layer(mp, pp, wg, bg, bl, wl1):
    ws = [wg, bg, bl, wl1]
    return pl.pallas_call(
        _layer_body,
        grid=(GRID,),
        in_specs=[pl.BlockSpec((BM, DPW), lambda i: (i, 0)),
                  pl.BlockSpec((BM, D_HID), lambda i: (i, 0))] + [_full(w) for w in ws],
        out_specs=[pl.BlockSpec((BM, D_HID), lambda i: (i, 0)),
                   pl.BlockSpec((BM, DPW), lambda i: (i, 0))],
        out_shape=[jax.ShapeDtypeStruct((NP, D_HID), jnp.float32),
                   jax.ShapeDtypeStruct((NP, DPW), jnp.float32)],
    )(mp, pp, *ws)


def _tc_pool(h, b3, wf, bf):
    return pl.pallas_call(
        _pool_body,
        grid=(GRID,),
        in_specs=[pl.BlockSpec((BM, D_HID), lambda i: (i, 0)),
                  pl.BlockSpec((1, 1, BM), lambda i: (i, 0, 0)),
                  _full(wf), _full(bf)],
        out_specs=pl.BlockSpec((N_GRAPHS, D_OUT), lambda i: (0, 0)),
        out_shape=jax.ShapeDtypeStruct((N_GRAPHS, D_OUT), jnp.float32),
        scratch_shapes=[pltpu.VMEM((N_GRAPHS, D_HID), jnp.float32)],
        compiler_params=pltpu.CompilerParams(
            dimension_semantics=("arbitrary",)),
    )(h, b3, wf, bf)


def _bucket_body(src_hbm, dst_hbm, plist, counts, sv0, dv0, sv1, dv1, ss, cs,
                 sem0, sem1):
    wid = lax.axis_index("c") * 16 + lax.axis_index("s")
    lo = wid * DPT
    hi = lo + DPT

    def issue(b, sv, dv, sem):
        pltpu.async_copy(src_hbm.at[pl.ds(b * EBLK, EBLK)], sv, sem)
        pltpu.async_copy(dst_hbm.at[pl.ds(b * EBLK, EBLK)], dv, sem)

    def wait(b, sv, dv, sem):
        pltpu.make_async_copy(src_hbm.at[pl.ds(b * EBLK, EBLK)], sv, sem).wait()
        pltpu.make_async_copy(dst_hbm.at[pl.ds(b * EBLK, EBLK)], dv, sem).wait()

    def process(sv, dv, gc):
        def ch(c, off):
            s16 = sv[pl.ds(c * 16, 16)]
            d16 = dv[pl.ds(c * 16, 16)]
            m = (d16 >= lo) & (d16 < hi)
            packed = (s16 << 9) | (d16 - lo)
            plsc.store_compressed(ss.at[pl.ds(off, 16)], packed, mask=m)
            pc = plsc.all_reduce_population_count(m)
            return off + pc[0]

        off = lax.fori_loop(0, EBLK // 16, ch, 0)
        ss[pl.ds(off, 16)] = jnp.full((16,), lo << 9, jnp.int32)
        offpad = ((off + 15) // 16) * 16
        base = pl.multiple_of(wid * CAP + gc, 16)
        pltpu.sync_copy(ss.at[pl.ds(0, EBLK + 16)],
                        plist.at[pl.ds(base, EBLK + 16)])
        return gc + offpad

    issue(0, sv0, dv0, sem0)

    def blkpair(p, gc):
        b0 = 2 * p
        issue(b0 + 1, sv1, dv1, sem1)
        wait(b0, sv0, dv0, sem0)
        gc = process(sv0, dv0, gc)

        @pl.when(b0 + 2 < NBLK_E)
        def _():
            issue(b0 + 2, sv0, dv0, sem0)

        wait(b0 + 1, sv1, dv1, sem1)
        gc = process(sv1, dv1, gc)
        return gc

    gc = lax.fori_loop(0, NBLK_E // 2, blkpair, 0)
    for k in range(256 // 16):
        ss[pl.ds(k * 16, 16)] = jnp.full((16,), lo << 9, jnp.int32)
    base = pl.multiple_of(wid * CAP + gc, 16)
    pltpu.sync_copy(ss.at[pl.ds(0, 256)], plist.at[pl.ds(base, 256)])
    padded = (gc // 256) * 256 + 256
    cs[...] = jnp.full((16,), padded, jnp.int32)
    pltpu.sync_copy(cs.at[pl.ds(0, 16)],
                    counts.at[pl.ds(pl.multiple_of(wid * 16, 16), 16)])


def _scmax_body(q_hbm, plist, counts, m_hbm, acc0, acc1, rows0, rows1, pbuf,
                sidx0, sidx1, dlb0, dlb1, cntv, sem0, sem1, sem2):
    wid = lax.axis_index("c") * 16 + lax.axis_index("s")
    lo = pl.multiple_of(wid * DPT, DPT)
    base = pl.multiple_of(wid * CAP, CHUNK)
    pltpu.sync_copy(q_hbm.at[pl.ds(lo, DPT)], acc0)
    pltpu.async_copy(q_hbm.at[pl.ds(lo, DPT)], acc1, sem2)
    pltpu.sync_copy(counts.at[pl.ds(pl.multiple_of(wid * 16, 16), 16)], cntv)
    n = cntv[pl.ds(0, 16)][0] // CHUNK

    def load_iblk(b):
        pltpu.sync_copy(plist.at[pl.ds(base + b * IBLK, IBLK)],
                        pbuf.at[pl.ds(0, IBLK)])

    def decode(c, sidx, dlb):
        o = (c % (IBLK // CHUNK)) * CHUNK
        for k in range(CHUNK // 16):
            v = pbuf[pl.ds(o + k * 16, 16)]
            sidx[pl.ds(k * 16, 16)] = v >> 9
            dlb[pl.ds(k * 16, 16)] = v & 511

    def rmw2(dl0, dl1, rows, e0):
        nj = DPW // 16

        def ld(j):
            sl = pl.ds(j * 16, 16)
            return (plsc.bitcast(acc0[dl0, sl], jnp.bfloat16),
                    plsc.bitcast(acc1[dl1, sl], jnp.bfloat16),
                    plsc.bitcast(rows[e0, sl], jnp.bfloat16),
                    plsc.bitcast(rows[e0 + 1, sl], jnp.bfloat16))

        prev = ld(0)
        for j in range(1, nj + 1):
            cur = ld(j) if j < nj else None
            a0, a1, r0, r1 = prev
            sl = pl.ds((j - 1) * 16, 16)
            acc0[dl0, sl] = plsc.bitcast(jnp.maximum(a0, r0), jnp.float32)
            acc1[dl1, sl] = plsc.bitcast(jnp.maximum(a1, r1), jnp.float32)
            prev = cur

    def process(rows, dlb):
        def grp(i, _):
            w = dlb[pl.ds(i * 16, 16)]
            for u in range(8):
                rmw2(w[2 * u], w[2 * u + 1], rows, i * 16 + 2 * u)
            return 0
        lax.fori_loop(0, CHUNK // 16, grp, 0)

    load_iblk(0)
    decode(0, sidx0, dlb0)
    pltpu.make_async_copy(q_hbm.at[pl.ds(lo, DPT)], acc1, sem2).wait()
    pltpu.async_copy(q_hbm.at[sidx0], rows0, sem0)

    def pair(p, _):
        c1 = 2 * p + 1
        decode(c1, sidx1, dlb1)
        pltpu.async_copy(q_hbm.at[sidx1], rows1, sem1)
        pltpu.make_async_copy(q_hbm.at[sidx0], rows0, sem0).wait()
        process(rows0, dlb0)
        c2 = 2 * p + 2

        @pl.when(c2 < n)
        def _():
            @pl.when(c2 % (IBLK // CHUNK) == 0)
            def _():
                load_iblk(c2 // (IBLK // CHUNK))
            decode(c2, sidx0, dlb0)
            pltpu.async_copy(q_hbm.at[sidx0], rows0, sem0)

        pltpu.make_async_copy(q_hbm.at[sidx1], rows1, sem1).wait()
        process(rows1, dlb1)
        return 0

    lax.fori_loop(0, n // 2, pair, 0)

    def mrow(r, _):
        for j in range(DPW // 16):
            sl = pl.ds(j * 16, 16)
            a = plsc.bitcast(acc0[r, sl], jnp.bfloat16)
            b = plsc.bitcast(acc1[r, sl], jnp.bfloat16)
            acc0[r, sl] = plsc.bitcast(jnp.maximum(a, b), jnp.float32)
        return 0

    lax.fori_loop(0, DPT, mrow, 0)
    pltpu.sync_copy(acc0, m_hbm.at[pl.ds(lo, DPT)])


@functools.cache
def _sc_kernels():
    mesh = plsc.VectorSubcoreMesh(core_axis_name="c", subcore_axis_name="s",
                                  num_cores=2, num_subcores=16)
    params = pltpu.CompilerParams(needs_layout_passes=False)
    bucket = pl.kernel(
        _bucket_body,
        out_type=(jax.ShapeDtypeStruct((NT * CAP,), jnp.int32),
                  jax.ShapeDtypeStruct((NT * 16,), jnp.int32)),
        mesh=mesh,
        compiler_params=params,
        scratch_types=[pltpu.VMEM((EBLK,), jnp.int32),
                       pltpu.VMEM((EBLK,), jnp.int32),
                       pltpu.VMEM((EBLK,), jnp.int32),
                       pltpu.VMEM((EBLK,), jnp.int32),
                       pltpu.VMEM((EBLK + 32,), jnp.int32),
                       pltpu.VMEM((16,), jnp.int32),
                       pltpu.SemaphoreType.DMA,
                       pltpu.SemaphoreType.DMA])
    scmax = pl.kernel(
        _scmax_body,
        out_type=jax.ShapeDtypeStruct((NP, DPW), jnp.float32),
        mesh=mesh,
        compiler_params=params,
        scratch_types=[pltpu.VMEM((DPT, DPW), jnp.float32),
                       pltpu.VMEM((DPT, DPW), jnp.float32),
                       pltpu.VMEM((CHUNK, DPW), jnp.float32),
                       pltpu.VMEM((CHUNK, DPW), jnp.float32),
                       pltpu.VMEM((IBLK + 16,), jnp.int32),
                       pltpu.VMEM((CHUNK,), jnp.int32),
                       pltpu.VMEM((CHUNK,), jnp.int32),
                       pltpu.VMEM((CHUNK + 16,), jnp.int32),
                       pltpu.VMEM((CHUNK + 16,), jnp.int32),
                       pltpu.VMEM((16,), jnp.int32),
                       pltpu.SemaphoreType.DMA,
                       pltpu.SemaphoreType.DMA,
                       pltpu.SemaphoreType.DMA])
    return bucket, scmax


def kernel(x, edge_index, batch, W_i1, b_i1, W_i2, b_i2, W_p1, b_p1,
           W_p2, b_p2, W_l, b_l, W_g, b_g, W_f, b_f):
    f32 = jnp.float32
    src = edge_index[0].astype(jnp.int32)
    dst = edge_index[1].astype(jnp.int32)
    xp = jnp.pad(x.astype(f32), ((0, NP - N_NODES), (0, 0)))
    bp = jnp.pad(batch.astype(jnp.int32), (0, NP - N_NODES),
                 constant_values=N_GRAPHS)

    wl1 = W_l[:D_HID]
    wl2 = jnp.pad(W_l[D_HID:], ((0, 128 - 3), (0, 0)))
    wp2 = jnp.pad(W_p2, ((0, 0), (0, 128 - 3)))
    bp2 = jnp.pad(b_p2, (0, 128 - 3)).reshape(1, 128)

    r = lambda v: v.reshape(1, -1)
    qp, pp = _tc_prep(xp, W_i1, r(b_i1), W_i2, r(b_i2), W_p1, r(b_p1),
                     wp2, bp2, wl2, wl1)
    bucket, scmax = _sc_kernels()
    plist, counts = bucket(src, dst)
    h = None
    for _ in range(3):
        mp = scmax(qp, plist, counts)
        h, qp = _tc_layer(mp, pp, W_g, r(b_g), r(b_l), wl1)
    out = _tc_pool(h, bp.reshape(GRID, 1, BM), W_f, r(b_f))
    return out
